# baseline (speedup 1.0000x reference)
import functools

import jax
import jax.numpy as jnp
from jax import lax
from jax.experimental import pallas as pl
from jax.experimental.pallas import tpu as pltpu
from jax.experimental.pallas import tpu_sc as plsc

B = 1024
EMB = 32
L_IDS = 50
L_CIDS = 20
N_IDS = 100000

NW = 32
CHUNK = 80

BT = 128
TN = 8192
NT = 13
NP = NT * TN

_HIGH = lax.Precision.HIGHEST


def _sc_gather(ids_flat, cids_flat, table_dae, table_cnn):
    n_dae = ids_flat.shape[0]
    n_cnn = cids_flat.shape[0]
    per_dae = n_dae // NW
    per_cnn = n_cnn // NW
    mesh = plsc.VectorSubcoreMesh(core_axis_name="c", subcore_axis_name="s")

    @functools.partial(
        pl.kernel,
        out_type=(
            jax.ShapeDtypeStruct((n_dae, EMB), jnp.float32),
            jax.ShapeDtypeStruct((n_cnn, EMB), jnp.float32),
        ),
        mesh=mesh,
        scratch_types=[
            pltpu.VMEM((per_dae,), jnp.int32),
            pltpu.VMEM((per_dae, EMB), jnp.float32),
            pltpu.SemaphoreType.DMA,
        ],
        compiler_params=pltpu.CompilerParams(use_tc_tiling_on_sc=False),
    )
    def gather_kernel(ids_hbm, cids_hbm, tdae_hbm, tcnn_hbm,
                      odae_hbm, ocnn_hbm, idx_v, rows_v, sem):
        wid = lax.axis_index("s") * 2 + lax.axis_index("c")

        base = wid * per_dae
        pltpu.sync_copy(ids_hbm.at[pl.ds(base, per_dae)], idx_v)

        @pl.loop(0, per_dae // CHUNK)
        def _(c):
            pltpu.async_copy(
                tdae_hbm.at[idx_v.at[pl.ds(c * CHUNK, CHUNK)]],
                rows_v.at[pl.ds(c * CHUNK, CHUNK)], sem)

        @pl.loop(0, per_dae // CHUNK)
        def _(c):
            pltpu.make_async_copy(
                tdae_hbm.at[idx_v.at[pl.ds(c * CHUNK, CHUNK)]],
                rows_v.at[pl.ds(c * CHUNK, CHUNK)], sem).wait()

        pltpu.sync_copy(rows_v, odae_hbm.at[pl.ds(base, per_dae)])

        base2 = wid * per_cnn
        pltpu.sync_copy(cids_hbm.at[pl.ds(base2, per_cnn)],
                        idx_v.at[pl.ds(0, per_cnn)])

        @pl.loop(0, per_cnn // CHUNK)
        def _(c):
            pltpu.async_copy(
                tcnn_hbm.at[idx_v.at[pl.ds(c * CHUNK, CHUNK)]],
                rows_v.at[pl.ds(c * CHUNK, CHUNK)], sem)

        @pl.loop(0, per_cnn // CHUNK)
        def _(c):
            pltpu.make_async_copy(
                tcnn_hbm.at[idx_v.at[pl.ds(c * CHUNK, CHUNK)]],
                rows_v.at[pl.ds(c * CHUNK, CHUNK)], sem).wait()

        pltpu.sync_copy(rows_v.at[pl.ds(0, per_cnn)],
                        ocnn_hbm.at[pl.ds(base2, per_cnn)])

    return gather_kernel(ids_flat, cids_flat, table_dae, table_cnn)


def _decode_body(we_ref, wf_ref, m_ref):
    m128 = lax.dot_general(we_ref[...], wf_ref[...],
                           (((0,), (0,)), ((), ())),
                           preferred_element_type=jnp.float32,
                           precision=_HIGH)
    m_ref[...] = (m128[0:32, 0:32] + m128[32:64, 32:64]
                  + m128[64:96, 64:96] + m128[96:128, 96:128])


def _decode(W_emb_dae, W_dae_ff1):
    return pl.pallas_call(
        _decode_body,
        out_shape=jax.ShapeDtypeStruct((EMB, EMB), jnp.float32),
    )(W_emb_dae.reshape(N_IDS // 4, 4 * EMB),
      W_dae_ff1.reshape(N_IDS // 4, 4 * EMB))


def _seg_sum(flat, length):
    sel = (lax.broadcasted_iota(jnp.int32, (length * EMB, EMB), 0) % EMB
           == lax.broadcasted_iota(jnp.int32, (length * EMB, EMB), 1)
           ).astype(jnp.float32)
    return jnp.dot(flat, sel, preferred_element_type=jnp.float32,
                   precision=_HIGH)


def _prep_body(gd_ref, gc_ref, m_ref, bd_ref, wc_ref, bc_ref,
               yd_ref, yc_ref):
    sd = _seg_sum(gd_ref[...], L_IDS)
    x = jnp.maximum(sd, 0.0)
    yd = jnp.dot(x, m_ref[...], preferred_element_type=jnp.float32,
                 precision=_HIGH) + bd_ref[...]
    yd_ref[...] = jnp.maximum(yd, 0.0)

    sc = _seg_sum(gc_ref[...], L_CIDS)
    c2 = jnp.dot(sc, wc_ref[...], preferred_element_type=jnp.float32,
                 precision=_HIGH) + bc_ref[...]
    c2 = jnp.maximum(c2, 0.0)
    cmax = jnp.max(c2, axis=1, keepdims=True)
    ce = jnp.exp(c2 - cmax)
    yc_ref[...] = ce / jnp.sum(ce, axis=1, keepdims=True)


def _prep(g_dae, g_cnn, m32, b_dae, W_cnn_ff1, b_cnn):
    return pl.pallas_call(
        _prep_body,
        out_shape=(
            jax.ShapeDtypeStruct((B, EMB), jnp.float32),
            jax.ShapeDtypeStruct((B, EMB), jnp.float32),
        ),
    )(g_dae, g_cnn, m32, b_dae, W_cnn_ff1, b_cnn)


NBT = B // BT
NBUF = 4
RG = 8
NGJ = BT // RG
W_TAIL = N_IDS - (NT - 1) * TN


def _head_body(h_ref, w_ref, b_ref, o_hbm, slab, mref, sref, mtile,
               stg, sems):
    i = pl.program_id(0)
    p = pl.program_id(1)
    j = pl.program_id(2)
    col0 = j * TN

    @pl.when((p == 0) & (j < NT))
    def _():
        z = jnp.dot(h_ref[...], w_ref[...],
                    preferred_element_type=jnp.float32)
        z = jnp.maximum(z + b_ref[...], 0.0)
        valid = (col0 + lax.broadcasted_iota(jnp.int32, (BT, TN), 1)) < N_IDS
        zm = jnp.where(valid, z, -3.0e38)
        tmax = jnp.max(zm, axis=1, keepdims=True)
        m_old = jnp.where(j == 0, -3.0e38, mref[:, 0:1])
        m_new = jnp.maximum(m_old, tmax)
        e = jnp.exp(zm - m_new)
        slab[:, pl.ds(col0, TN)] = e.astype(jnp.bfloat16)
        ts = jnp.sum(e, axis=1, keepdims=True)
        s_old = jnp.where(j == 0, 0.0, sref[:, 0:1])
        sref[:, 0:1] = s_old * jnp.exp(m_old - m_new) + ts
        mref[:, 0:1] = m_new
        mtile[:, pl.ds(j * 128, 128)] = jnp.broadcast_to(m_new, (BT, 128))

    @pl.when(p == 1)
    def _():
        buf = lax.rem(j, NBUF)
        rows = pl.ds(j * RG, RG)

        @pl.when(j >= NBUF)
        def _():
            pltpu.make_async_copy(
                stg.at[buf],
                o_hbm.at[pl.ds(i * BT + (j - NBUF) * RG, RG)],
                sems.at[buf]).wait()

        @pl.when((i > 0) & (j < NBUF))
        def _():
            pltpu.make_async_copy(
                stg.at[buf],
                o_hbm.at[pl.ds((i - 1) * BT + (NGJ - NBUF + j) * RG, RG)],
                sems.at[buf]).wait()

        m_fin = mref[rows, 0:1]
        inv_s = 1.0 / sref[rows, 0:1]
        for jj in range(NT):
            width = TN if jj < NT - 1 else W_TAIL
            m_j = mtile[rows, jj * 128:jj * 128 + 1]
            corr = jnp.exp(m_j - m_fin) * inv_s
            ev = slab[rows, jj * TN:jj * TN + width].astype(jnp.float32)
            stg[buf, :, jj * TN:jj * TN + width] = ev * corr

        pltpu.make_async_copy(
            stg.at[buf],
            o_hbm.at[pl.ds(i * BT + j * RG, RG)],
            sems.at[buf]).start()

        @pl.when((i == NBT - 1) & (j == NGJ - 1))
        def _():
            for g_ in range(NGJ - NBUF, NGJ):
                pltpu.make_async_copy(
                    stg.at[g_ % NBUF],
                    o_hbm.at[pl.ds((NBT - 1) * BT + g_ * RG, RG)],
                    sems.at[g_ % NBUF]).wait()


def _head(h, W_bf, b_ff2):
    grid = (NBT, 2, NGJ)
    return pl.pallas_call(
        _head_body,
        grid=grid,
        in_specs=[
            pl.BlockSpec((BT, 64), lambda i, p, j: (i, 0)),
            pl.BlockSpec((64, TN),
                         lambda i, p, j: (0, jnp.minimum(j, NT - 1) * (1 - p))),
            pl.BlockSpec((1, TN),
                         lambda i, p, j: (0, jnp.minimum(j, NT - 1) * (1 - p))),
        ],
        out_specs=pl.BlockSpec(memory_space=pl.ANY),
        out_shape=jax.ShapeDtypeStruct((B, N_IDS), jnp.float32),
        scratch_shapes=[
            pltpu.VMEM((BT, NP), jnp.bfloat16),
            pltpu.VMEM((BT, 128), jnp.float32),
            pltpu.VMEM((BT, 128), jnp.float32),
            pltpu.VMEM((BT, NT * 128), jnp.float32),
            pltpu.VMEM((NBUF, RG, N_IDS), jnp.float32),
            pltpu.SemaphoreType.DMA((NBUF,)),
        ],
        compiler_params=pltpu.CompilerParams(
            dimension_semantics=("arbitrary", "arbitrary", "arbitrary"),
        ),
    )(h, W_bf, b_ff2)


def kernel(ids, cids, W_emb_dae, W_dae_ff1, b_dae_ff1, W_emb_cnn,
           W_cnn_ff1, b_cnn_ff1, W_ff, b_ff):
    ids_flat = ids.reshape(-1).astype(jnp.int32)
    cids_flat = cids.reshape(-1).astype(jnp.int32)

    g_dae, g_cnn = _sc_gather(ids_flat, cids_flat, W_emb_dae, W_emb_cnn)
    m32 = _decode(W_emb_dae, W_dae_ff1)

    y_dae, y_cnn = _prep(
        g_dae.reshape(B, L_IDS * EMB),
        g_cnn.reshape(B, L_CIDS * EMB),
        m32,
        b_dae_ff1.reshape(1, EMB),
        W_cnn_ff1,
        b_cnn_ff1.reshape(1, EMB),
    )
    h = jnp.concatenate([y_dae, y_cnn], axis=1).astype(jnp.bfloat16)
    return _head(h, W_ff.astype(jnp.bfloat16), b_ff.reshape(1, N_IDS))
